# baseline (speedup 1.0000x reference)
import functools
import math

import jax
import jax.numpy as jnp
from jax import lax
from jax.experimental import pallas as pl
from jax.experimental.pallas import tpu as pltpu
from jax.experimental.pallas import tpu_sc as plsc

D_MODEL = 64
SCALE = math.sqrt(D_MODEL)
LANES = 16
NC, NS = 2, 16
NW = NC * NS
TBATCH = 128
K = 4
GROUP = K * TBATCH


def _make_kernel(n_idx: int):
    assert n_idx % (NW * 2 * GROUP) == 0
    per_w = n_idx // NW
    ntrans = per_w // TBATCH
    ngroups = per_w // GROUP
    mesh = plsc.VectorSubcoreMesh(core_axis_name="c", subcore_axis_name="s")

    @functools.partial(
        pl.kernel,
        out_type=jax.ShapeDtypeStruct((n_idx, D_MODEL), jnp.float32),
        mesh=mesh,
        scratch_types=[
            pltpu.VMEM((ntrans, TBATCH), jnp.int32),
            pltpu.VMEM((2, GROUP, D_MODEL), jnp.float32),
            pltpu.SemaphoreType.DMA,
            pltpu.SemaphoreType.DMA,
        ],
        compiler_params=pltpu.CompilerParams(use_tc_tiling_on_sc=False),
    )
    def emb_kernel(x_hbm, table_hbm, out_hbm, idx_v, rows_v, sem0, sem1):
        wid = lax.axis_index("s") * NC + lax.axis_index("c")
        base = wid * per_w
        sems = (sem0, sem1)

        pltpu.sync_copy(x_hbm.at[wid], idx_v)

        def fire(group, slot, sem):
            for t in range(K):
                pltpu.async_copy(
                    table_hbm.at[idx_v.at[group * K + t]],
                    rows_v.at[slot].at[pl.ds(t * TBATCH, TBATCH)],
                    sem,
                )

        def drain(group, slot, sem):
            for t in range(K):
                pltpu.make_async_copy(
                    table_hbm.at[idx_v.at[group * K + t]],
                    rows_v.at[slot].at[pl.ds(t * TBATCH, TBATCH)],
                    sem,
                ).wait()

        fire(0, 0, sem0)

        @pl.loop(0, ngroups, step=2)
        def group_loop(g):
            for b in range(2):
                cur = g + b
                nxt = 1 - b

                @pl.when(cur + 1 < ngroups)
                def _start_next():
                    fire(cur + 1, nxt, sems[nxt])

                drain(cur, b, sems[b])

                @plsc.parallel_loop(0, GROUP, unroll=4)
                def _scale(r):
                    for j in range(D_MODEL // LANES):
                        sl = pl.ds(j * LANES, LANES)
                        rows_v[b, r, sl] = rows_v[b, r, sl] * SCALE

                pltpu.sync_copy(
                    rows_v.at[b],
                    out_hbm.at[pl.ds(base + cur * GROUP, GROUP)],
                )

    return emb_kernel


def kernel(x, table):
    b, l = x.shape
    n_idx = b * l
    x_flat = x.reshape(NW, n_idx // (NW * TBATCH), TBATCH).astype(jnp.int32)
    out = _make_kernel(n_idx)(x_flat, table)
    return out.reshape(b, l, D_MODEL)
